# Initial kernel scaffold; baseline (speedup 1.0000x reference)
import functools

import jax
import jax.numpy as jnp
from jax import lax
from jax.experimental import pallas as pl
from jax.experimental.pallas import tpu as pltpu
from jax.experimental.pallas import tpu_sc as plsc

N0 = 10000
E0 = 320000
FD = 128
SEG = 128

NCORE = 2
NSUB = 16
NTILES = NCORE * NSUB

NP = 10240
ROWS_PER_SUB = NP // NSUB

EC = 128
KE = 79
EP = NTILES * KE * EC

RPT = NP // NTILES
PC = 64
KP = RPT // PC
SEGP = 136

_mesh = plsc.VectorSubcoreMesh(core_axis_name="c", subcore_axis_name="s")


def _mish(v):
    sp = jnp.maximum(v, 0.0) + jnp.log1p(jnp.exp(-jnp.abs(v)))
    return v * jnp.tanh(sp)


def _deg_body(dst_hbm, ones_hbm, zeros_hbm, degp_hbm, idx_v, ones_v, deg_sp):
    c = lax.axis_index("c")
    s = lax.axis_index("s")
    w = c * NSUB + s
    pltpu.sync_copy(dst_hbm.at[w], idx_v)
    pltpu.sync_copy(ones_hbm, ones_v)
    pltpu.sync_copy(zeros_hbm, deg_sp.at[pl.ds(s * ROWS_PER_SUB, ROWS_PER_SUB)])
    plsc.subcore_barrier()

    @pl.loop(0, KE)
    def _(j):
        pltpu.sync_copy(ones_v, deg_sp.at[idx_v.at[j]], add=True)

    plsc.subcore_barrier()
    pltpu.sync_copy(
        deg_sp.at[pl.ds(s * ROWS_PER_SUB, ROWS_PER_SUB)],
        degp_hbm.at[c, pl.ds(s * ROWS_PER_SUB, ROWS_PER_SUB)],
    )


_deg_call = functools.partial(
    pl.kernel,
    out_type=jax.ShapeDtypeStruct((NCORE, NP, 16), jnp.float32),
    mesh=_mesh,
    scratch_types=[
        pltpu.VMEM((KE, EC), jnp.int32),
        pltpu.VMEM((EC, 16), jnp.float32),
        pltpu.VMEM_SHARED((NP, 16), jnp.float32),
    ],
)(_deg_body)


def _edge_body(h_hbm, src_hbm, dst_hbm, zeros_hbm, accp_hbm,
               srcv, dstv, rows, sem, acc_sp):
    c = lax.axis_index("c")
    s = lax.axis_index("s")
    w = c * NSUB + s
    pltpu.sync_copy(src_hbm.at[w], srcv)
    pltpu.sync_copy(dst_hbm.at[w], dstv)
    pltpu.sync_copy(zeros_hbm, acc_sp.at[pl.ds(s * ROWS_PER_SUB, ROWS_PER_SUB)])
    plsc.subcore_barrier()

    @pl.loop(0, KE)
    def _(j):
        pltpu.async_copy(h_hbm.at[srcv.at[j]], rows, sem).wait()
        pltpu.sync_copy(rows, acc_sp.at[dstv.at[j]], add=True)

    plsc.subcore_barrier()
    pltpu.sync_copy(
        acc_sp.at[pl.ds(s * ROWS_PER_SUB, ROWS_PER_SUB)],
        accp_hbm.at[c, pl.ds(s * ROWS_PER_SUB, ROWS_PER_SUB)],
    )


_edge_call = functools.partial(
    pl.kernel,
    out_type=jax.ShapeDtypeStruct((NCORE, NP, FD), jnp.float32),
    mesh=_mesh,
    scratch_types=[
        pltpu.VMEM((KE, EC), jnp.int32),
        pltpu.VMEM((KE, EC), jnp.int32),
        pltpu.VMEM((EC, FD), jnp.float32),
        pltpu.SemaphoreType.DMA,
        pltpu.VMEM_SHARED((NP, FD), jnp.float32),
    ],
)(_edge_body)


def _pool_body(x1_hbm, b_hbm, ones_hbm, zsum_hbm, zcnt_hbm,
               sums_hbm, cnt_hbm, bidx, xrows, ones_v, sums_sp, cnt_sp):
    c = lax.axis_index("c")
    s = lax.axis_index("s")
    w = c * NSUB + s
    pltpu.sync_copy(b_hbm.at[w], bidx)
    pltpu.sync_copy(x1_hbm.at[pl.ds(w * RPT, RPT)], xrows)
    pltpu.sync_copy(ones_hbm, ones_v)

    @pl.when(s == 0)
    def _():
        pltpu.sync_copy(zsum_hbm, sums_sp)
        pltpu.sync_copy(zcnt_hbm, cnt_sp)

    plsc.subcore_barrier()

    @pl.loop(0, KP)
    def _(j):
        pltpu.sync_copy(xrows.at[pl.ds(j * PC, PC)], sums_sp.at[bidx.at[j]],
                        add=True)
        pltpu.sync_copy(ones_v, cnt_sp.at[bidx.at[j]], add=True)

    plsc.subcore_barrier()

    @pl.when(s == 0)
    def _():
        pltpu.sync_copy(sums_sp, sums_hbm.at[c])
        pltpu.sync_copy(cnt_sp, cnt_hbm.at[c])


_pool_call = functools.partial(
    pl.kernel,
    out_type=(
        jax.ShapeDtypeStruct((NCORE, SEGP, FD), jnp.float32),
        jax.ShapeDtypeStruct((NCORE, SEGP, 16), jnp.float32),
    ),
    mesh=_mesh,
    scratch_types=[
        pltpu.VMEM((KP, PC), jnp.int32),
        pltpu.VMEM((RPT, FD), jnp.float32),
        pltpu.VMEM((PC, 16), jnp.float32),
        pltpu.VMEM_SHARED((SEGP, FD), jnp.float32),
        pltpu.VMEM_SHARED((SEGP, 16), jnp.float32),
    ],
)(_pool_body)


_ROWB = 512


def _dinv_from(degp_ref):
    d = degp_ref[0, :, 0:1] + degp_ref[1, :, 0:1] + 1.0
    return lax.rsqrt(d)


def _t1_body(x_ref, degp_ref, w1_ref, h_ref):
    h = lax.dot_general(x_ref[...], w1_ref[...], (((1,), (1,)), ((), ())),
                        preferred_element_type=jnp.float32)
    h_ref[...] = h * _dinv_from(degp_ref)


def _t2_body(accp_ref, hp_ref, degp_ref, w2_ref, b1_ref, out_ref):
    dinv = _dinv_from(degp_ref)
    agg = dinv * (accp_ref[0] + accp_ref[1] + hp_ref[...]) + b1_ref[...]
    o1 = _mish(agg)
    h2 = lax.dot_general(o1, w2_ref[...], (((1,), (1,)), ((), ())),
                         preferred_element_type=jnp.float32)
    out_ref[...] = h2 * dinv


def _t3_body(accp_ref, hp_ref, degp_ref, wfc_ref, b2_ref, bfc_ref, x1_ref):
    dinv = _dinv_from(degp_ref)
    agg = dinv * (accp_ref[0] + accp_ref[1] + hp_ref[...]) + b2_ref[...]
    o2 = _mish(agg)
    fc = lax.dot_general(o2, wfc_ref[...], (((1,), (1,)), ((), ())),
                         preferred_element_type=jnp.float32)
    x1_ref[...] = _mish(fc + bfc_ref[...])


def _t4_body(sums_ref, cnt_ref, g_ref, bt_ref, x2_ref):
    ssum = sums_ref[0, 0:SEG, :] + sums_ref[1, 0:SEG, :]
    cnt = cnt_ref[0, 0:SEG, 0:1] + cnt_ref[1, 0:SEG, 0:1]
    pooled = ssum / jnp.maximum(cnt, 1.0)
    mu = jnp.mean(pooled, axis=0, keepdims=True)
    var = jnp.mean((pooled - mu) ** 2, axis=0, keepdims=True)
    x2_ref[...] = (pooled - mu) * lax.rsqrt(var + 1e-5) * g_ref[...] + bt_ref[...]


_GRID = NP // _ROWB
_row_spec = pl.BlockSpec((_ROWB, FD), lambda i: (i, 0))
_degp_spec = pl.BlockSpec((NCORE, _ROWB, 16), lambda i: (0, i, 0))
_accp_spec = pl.BlockSpec((NCORE, _ROWB, FD), lambda i: (0, i, 0))
_w_spec = pl.BlockSpec((FD, FD), lambda i: (0, 0))
_b_spec = pl.BlockSpec((1, FD), lambda i: (0, 0))

_t1_call = pl.pallas_call(
    _t1_body,
    grid=(_GRID,),
    in_specs=[_row_spec, _degp_spec, _w_spec],
    out_specs=_row_spec,
    out_shape=jax.ShapeDtypeStruct((NP, FD), jnp.float32),
)

_t2_call = pl.pallas_call(
    _t2_body,
    grid=(_GRID,),
    in_specs=[_accp_spec, _row_spec, _degp_spec, _w_spec, _b_spec],
    out_specs=_row_spec,
    out_shape=jax.ShapeDtypeStruct((NP, FD), jnp.float32),
)

_t3_call = pl.pallas_call(
    _t3_body,
    grid=(_GRID,),
    in_specs=[_accp_spec, _row_spec, _degp_spec, _w_spec, _b_spec, _b_spec],
    out_specs=_row_spec,
    out_shape=jax.ShapeDtypeStruct((NP, FD), jnp.float32),
)

_t4_call = pl.pallas_call(
    _t4_body,
    in_specs=[
        pl.BlockSpec((NCORE, SEGP, FD), lambda: (0, 0, 0)),
        pl.BlockSpec((NCORE, SEGP, 16), lambda: (0, 0, 0)),
        pl.BlockSpec((1, FD), lambda: (0, 0)),
        pl.BlockSpec((1, FD), lambda: (0, 0)),
    ],
    out_specs=pl.BlockSpec((SEG, FD), lambda: (0, 0)),
    out_shape=jax.ShapeDtypeStruct((SEG, FD), jnp.float32),
)


def kernel(x, edge_index, batch, W1, b1, W2, b2, Wfc, bfc, gamma, beta):
    f32 = jnp.float32
    i32 = jnp.int32

    ei = edge_index.astype(i32)
    pad_e = jnp.full((EP - E0,), N0, i32)
    src_r = jnp.concatenate([ei[0], pad_e]).reshape(NTILES, KE, EC)
    dst_r = jnp.concatenate([ei[1], pad_e]).reshape(NTILES, KE, EC)

    bt_r = jnp.concatenate(
        [batch.astype(i32), jnp.full((NP - N0,), SEG, i32)]
    ).reshape(NTILES, KP, PC)

    xp = jnp.pad(x, ((0, NP - N0), (0, 0)))

    ones_ec = jnp.ones((EC, 16), f32)
    ones_pc = jnp.ones((PC, 16), f32)
    zeros_deg = jnp.zeros((ROWS_PER_SUB, 16), f32)
    zeros_rows = jnp.zeros((ROWS_PER_SUB, FD), f32)
    zeros_sum = jnp.zeros((SEGP, FD), f32)
    zeros_cnt = jnp.zeros((SEGP, 16), f32)

    degp = _deg_call(dst_r, ones_ec, zeros_deg)

    h1p = _t1_call(xp, degp, W1)
    acc1 = _edge_call(h1p, src_r, dst_r, zeros_rows)
    h2p = _t2_call(acc1, h1p, degp, W2, b1.reshape(1, FD))
    acc2 = _edge_call(h2p, src_r, dst_r, zeros_rows)
    x1p = _t3_call(acc2, h2p, degp, Wfc, b2.reshape(1, FD), bfc.reshape(1, FD))

    sums, cnt = _pool_call(x1p, bt_r, ones_pc, zeros_sum, zeros_cnt)
    x2 = _t4_call(sums, cnt, gamma.reshape(1, FD), beta.reshape(1, FD))

    return (x1p[:N0], x2)


# speedup vs baseline: 11.4961x; 11.4961x over previous
import functools

import jax
import jax.numpy as jnp
from jax import lax
from jax.experimental import pallas as pl
from jax.experimental.pallas import tpu as pltpu
from jax.experimental.pallas import tpu_sc as plsc

N0 = 10000
E0 = 320000
FD = 128
SEG = 128

NCORE = 2
NSUB = 16
NTILES = NCORE * NSUB

NP = 10240
ROWS_PER_SUB = NP // NSUB

EC = 128
KE = 79
EP = NTILES * KE * EC

RPT = NP // NTILES
PC = 64
KP = RPT // PC
SEGP = 136

_mesh = plsc.VectorSubcoreMesh(core_axis_name="c", subcore_axis_name="s")


def _mish(v):
    sp = jnp.maximum(v, 0.0) + jnp.log1p(jnp.exp(-jnp.abs(v)))
    return v * jnp.tanh(sp)


def _deg_body(dst_hbm, ones_hbm, zeros_hbm, degp_hbm, idx_v, ones_v, deg_sp):
    c = lax.axis_index("c")
    s = lax.axis_index("s")
    w = c * NSUB + s
    pltpu.sync_copy(dst_hbm.at[w], idx_v)
    pltpu.sync_copy(ones_hbm, ones_v)
    pltpu.sync_copy(zeros_hbm, deg_sp.at[pl.ds(s * ROWS_PER_SUB, ROWS_PER_SUB)])
    plsc.subcore_barrier()

    @pl.loop(0, KE)
    def _(j):
        pltpu.sync_copy(ones_v, deg_sp.at[idx_v.at[j]], add=True)

    plsc.subcore_barrier()
    pltpu.sync_copy(
        deg_sp.at[pl.ds(s * ROWS_PER_SUB, ROWS_PER_SUB)],
        degp_hbm.at[c, pl.ds(s * ROWS_PER_SUB, ROWS_PER_SUB)],
    )


_deg_call = functools.partial(
    pl.kernel,
    out_type=jax.ShapeDtypeStruct((NCORE, NP, FD), jnp.float32),
    mesh=_mesh,
    scratch_types=[
        pltpu.VMEM((KE, EC), jnp.int32),
        pltpu.VMEM((EC, FD), jnp.float32),
        pltpu.VMEM_SHARED((NP, FD), jnp.float32),
    ],
)(_deg_body)


def _edge_body(h_hbm, src_hbm, dst_hbm, zeros_hbm, accp_hbm,
               srcv, dstv, rows, sem, acc_sp):
    c = lax.axis_index("c")
    s = lax.axis_index("s")
    w = c * NSUB + s
    pltpu.sync_copy(src_hbm.at[w], srcv)
    pltpu.sync_copy(dst_hbm.at[w], dstv)
    pltpu.sync_copy(zeros_hbm, acc_sp.at[pl.ds(s * ROWS_PER_SUB, ROWS_PER_SUB)])
    plsc.subcore_barrier()

    @pl.loop(0, KE)
    def _(j):
        pltpu.async_copy(h_hbm.at[srcv.at[j]], rows, sem).wait()
        pltpu.sync_copy(rows, acc_sp.at[dstv.at[j]], add=True)

    plsc.subcore_barrier()
    pltpu.sync_copy(
        acc_sp.at[pl.ds(s * ROWS_PER_SUB, ROWS_PER_SUB)],
        accp_hbm.at[c, pl.ds(s * ROWS_PER_SUB, ROWS_PER_SUB)],
    )


_edge_call = functools.partial(
    pl.kernel,
    out_type=jax.ShapeDtypeStruct((NCORE, NP, FD), jnp.float32),
    mesh=_mesh,
    scratch_types=[
        pltpu.VMEM((KE, EC), jnp.int32),
        pltpu.VMEM((KE, EC), jnp.int32),
        pltpu.VMEM((EC, FD), jnp.float32),
        pltpu.SemaphoreType.DMA,
        pltpu.VMEM_SHARED((NP, FD), jnp.float32),
    ],
)(_edge_body)


def _pool_body(x1_hbm, b_hbm, ones_hbm, zsum_hbm, zcnt_hbm,
               sums_hbm, cnt_hbm, bidx, xrows, ones_v, sums_sp, cnt_sp):
    c = lax.axis_index("c")
    s = lax.axis_index("s")
    w = c * NSUB + s
    pltpu.sync_copy(b_hbm.at[w], bidx)
    pltpu.sync_copy(x1_hbm.at[pl.ds(w * RPT, RPT)], xrows)
    pltpu.sync_copy(ones_hbm, ones_v)

    @pl.when(s == 0)
    def _():
        pltpu.sync_copy(zsum_hbm, sums_sp)
        pltpu.sync_copy(zcnt_hbm, cnt_sp)

    plsc.subcore_barrier()

    @pl.loop(0, KP)
    def _(j):
        pltpu.sync_copy(xrows.at[pl.ds(j * PC, PC)], sums_sp.at[bidx.at[j]],
                        add=True)
        pltpu.sync_copy(ones_v, cnt_sp.at[bidx.at[j]], add=True)

    plsc.subcore_barrier()

    @pl.when(s == 0)
    def _():
        pltpu.sync_copy(sums_sp, sums_hbm.at[c])
        pltpu.sync_copy(cnt_sp, cnt_hbm.at[c])


_pool_call = functools.partial(
    pl.kernel,
    out_type=(
        jax.ShapeDtypeStruct((NCORE, SEGP, FD), jnp.float32),
        jax.ShapeDtypeStruct((NCORE, SEGP, FD), jnp.float32),
    ),
    mesh=_mesh,
    scratch_types=[
        pltpu.VMEM((KP, PC), jnp.int32),
        pltpu.VMEM((RPT, FD), jnp.float32),
        pltpu.VMEM((PC, FD), jnp.float32),
        pltpu.VMEM_SHARED((SEGP, FD), jnp.float32),
        pltpu.VMEM_SHARED((SEGP, FD), jnp.float32),
    ],
)(_pool_body)


_ROWB = 512


def _dinv_from(degp_ref):
    d = degp_ref[0, :, 0:1] + degp_ref[1, :, 0:1] + 1.0
    return lax.rsqrt(d)


def _t1_body(x_ref, degp_ref, w1_ref, h_ref):
    h = lax.dot_general(x_ref[...], w1_ref[...], (((1,), (1,)), ((), ())),
                        preferred_element_type=jnp.float32)
    h_ref[...] = h * _dinv_from(degp_ref)


def _t2_body(accp_ref, hp_ref, degp_ref, w2_ref, b1_ref, out_ref):
    dinv = _dinv_from(degp_ref)
    agg = dinv * (accp_ref[0] + accp_ref[1] + hp_ref[...]) + b1_ref[...]
    o1 = _mish(agg)
    h2 = lax.dot_general(o1, w2_ref[...], (((1,), (1,)), ((), ())),
                         preferred_element_type=jnp.float32)
    out_ref[...] = h2 * dinv


def _t3_body(accp_ref, hp_ref, degp_ref, wfc_ref, b2_ref, bfc_ref, x1_ref):
    dinv = _dinv_from(degp_ref)
    agg = dinv * (accp_ref[0] + accp_ref[1] + hp_ref[...]) + b2_ref[...]
    o2 = _mish(agg)
    fc = lax.dot_general(o2, wfc_ref[...], (((1,), (1,)), ((), ())),
                         preferred_element_type=jnp.float32)
    x1_ref[...] = _mish(fc + bfc_ref[...])


def _t4_body(sums_ref, cnt_ref, g_ref, bt_ref, x2_ref):
    ssum = sums_ref[0, 0:SEG, :] + sums_ref[1, 0:SEG, :]
    cnt = cnt_ref[0, 0:SEG, 0:1] + cnt_ref[1, 0:SEG, 0:1]
    pooled = ssum / jnp.maximum(cnt, 1.0)
    mu = jnp.mean(pooled, axis=0, keepdims=True)
    var = jnp.mean((pooled - mu) ** 2, axis=0, keepdims=True)
    x2_ref[...] = (pooled - mu) * lax.rsqrt(var + 1e-5) * g_ref[...] + bt_ref[...]


_GRID = NP // _ROWB
_row_spec = pl.BlockSpec((_ROWB, FD), lambda i: (i, 0))
_degp_spec = pl.BlockSpec((NCORE, _ROWB, FD), lambda i: (0, i, 0))
_accp_spec = pl.BlockSpec((NCORE, _ROWB, FD), lambda i: (0, i, 0))
_w_spec = pl.BlockSpec((FD, FD), lambda i: (0, 0))
_b_spec = pl.BlockSpec((1, FD), lambda i: (0, 0))

_t1_call = pl.pallas_call(
    _t1_body,
    grid=(_GRID,),
    in_specs=[_row_spec, _degp_spec, _w_spec],
    out_specs=_row_spec,
    out_shape=jax.ShapeDtypeStruct((NP, FD), jnp.float32),
)

_t2_call = pl.pallas_call(
    _t2_body,
    grid=(_GRID,),
    in_specs=[_accp_spec, _row_spec, _degp_spec, _w_spec, _b_spec],
    out_specs=_row_spec,
    out_shape=jax.ShapeDtypeStruct((NP, FD), jnp.float32),
)

_t3_call = pl.pallas_call(
    _t3_body,
    grid=(_GRID,),
    in_specs=[_accp_spec, _row_spec, _degp_spec, _w_spec, _b_spec, _b_spec],
    out_specs=_row_spec,
    out_shape=jax.ShapeDtypeStruct((NP, FD), jnp.float32),
)

_t4_call = pl.pallas_call(
    _t4_body,
    in_specs=[
        pl.BlockSpec((NCORE, SEGP, FD), lambda: (0, 0, 0)),
        pl.BlockSpec((NCORE, SEGP, FD), lambda: (0, 0, 0)),
        pl.BlockSpec((1, FD), lambda: (0, 0)),
        pl.BlockSpec((1, FD), lambda: (0, 0)),
    ],
    out_specs=pl.BlockSpec((SEG, FD), lambda: (0, 0)),
    out_shape=jax.ShapeDtypeStruct((SEG, FD), jnp.float32),
)


def kernel(x, edge_index, batch, W1, b1, W2, b2, Wfc, bfc, gamma, beta):
    f32 = jnp.float32
    i32 = jnp.int32

    ei = edge_index.astype(i32)
    pad_e = jnp.full((EP - E0,), N0, i32)
    src_r = jnp.concatenate([ei[0], pad_e]).reshape(NTILES, KE, EC)
    dst_r = jnp.concatenate([ei[1], pad_e]).reshape(NTILES, KE, EC)

    bt_r = jnp.concatenate(
        [batch.astype(i32), jnp.full((NP - N0,), SEG, i32)]
    ).reshape(NTILES, KP, PC)

    xp = jnp.pad(x, ((0, NP - N0), (0, 0)))

    ones_ec = jnp.ones((EC, FD), f32)
    ones_pc = jnp.ones((PC, FD), f32)
    zeros_rows = jnp.zeros((ROWS_PER_SUB, FD), f32)
    zeros_sum = jnp.zeros((SEGP, FD), f32)
    zeros_cnt = jnp.zeros((SEGP, FD), f32)

    degp = _deg_call(dst_r, ones_ec, zeros_rows)

    h1p = _t1_call(xp, degp, W1)
    acc1 = _edge_call(h1p, src_r, dst_r, zeros_rows)
    h2p = _t2_call(acc1, h1p, degp, W2, b1.reshape(1, FD))
    acc2 = _edge_call(h2p, src_r, dst_r, zeros_rows)
    x1p = _t3_call(acc2, h2p, degp, Wfc, b2.reshape(1, FD), bfc.reshape(1, FD))

    sums, cnt = _pool_call(x1p, bt_r, ones_pc, zeros_sum, zeros_cnt)
    x2 = _t4_call(sums, cnt, gamma.reshape(1, FD), beta.reshape(1, FD))

    return (x1p[:N0], x2)
